# Initial kernel scaffold; baseline (speedup 1.0000x reference)
import jax
import jax.numpy as jnp
from jax.experimental import pallas as pl


def kernel(x, edge_index, W1, b1, gamma, beta, W2, b2, Wmu, bmu, Wls, bls):
    raise NotImplementedError("write your pallas kernel here")


# speedup vs baseline: 17.2780x; 17.2780x over previous
import jax
import jax.numpy as jnp
from jax import lax
from jax.experimental import pallas as pl
from jax.experimental.pallas import tpu as pltpu
from jax.experimental.pallas import tpu_sc as plsc

N = 10000
DIN = 128
DH = 256
NPAD = 10112
RPT = NPAD // 16
E = 320000
CH = 128
EPAD = 327680
C1 = EPAD // (32 * CH)
C3 = EPAD // (16 * CH)
IB1 = 16
IB3 = 40

_f32 = jnp.float32


def _mesh():
    return plsc.VectorSubcoreMesh(core_axis_name="c", subcore_axis_name="s")


DE = 144


def _gin_body(xe_hbm, srcb, dstb, zr, agg_out,
              src_v, dst_v, buf, acc, sem, sem2):
    c = lax.axis_index("c")
    s = lax.axis_index("s")
    wid = s * 2 + c
    base = s * RPT
    pltpu.sync_copy(zr, acc.at[pl.ds(base, RPT)])
    plsc.subcore_barrier()

    def block(bk, carry):
        pltpu.sync_copy(srcb.at[wid, pl.ds(bk * IB1, IB1)], src_v)
        pltpu.sync_copy(dstb.at[wid, pl.ds(bk * IB1, IB1)], dst_v)

        def step(i, carry2):
            pltpu.async_copy(xe_hbm.at[src_v.at[i]], buf, sem).wait()
            pltpu.async_copy(buf, acc.at[dst_v.at[i]], sem2,
                             add=True).wait()
            return carry2

        lax.fori_loop(0, IB1, step, 0)
        return carry

    lax.fori_loop(0, C1 // IB1, block, 0)
    plsc.subcore_barrier()
    pltpu.sync_copy(acc.at[pl.ds(base, RPT)], agg_out.at[c, pl.ds(base, RPT)])


_gin_call = pl.kernel(
    _gin_body,
    out_type=jax.ShapeDtypeStruct((2, NPAD, DE), _f32),
    mesh=_mesh(),
    scratch_types=[
        pltpu.VMEM((IB1, CH), jnp.int32),
        pltpu.VMEM((IB1, CH), jnp.int32),
        pltpu.VMEM((CH, DE), _f32),
        pltpu.VMEM_SHARED((NPAD, DE), _f32),
        pltpu.SemaphoreType.DMA,
        pltpu.SemaphoreType.DMA,
    ],
    compiler_params=pltpu.CompilerParams(use_tc_tiling_on_sc=False),
)


def _gcn_body(y_hbm, srcb, dstb, zr, s_out,
              src_v, dst_v, buf, acc, sem, sem2):
    c = lax.axis_index("c")
    s = lax.axis_index("s")
    base = s * RPT
    pltpu.sync_copy(zr, acc.at[pl.ds(base, RPT)])
    plsc.subcore_barrier()

    def block(bk, carry):
        pltpu.sync_copy(srcb.at[c, s, pl.ds(bk * IB3, IB3)], src_v)
        pltpu.sync_copy(dstb.at[s, pl.ds(bk * IB3, IB3)], dst_v)

        def group(g, carry2):
            i0 = g * 2
            d0 = pltpu.async_copy(y_hbm.at[src_v.at[i0]],
                                  buf.at[pl.ds(0, CH)], sem)
            d1 = pltpu.async_copy(y_hbm.at[src_v.at[i0 + 1]],
                                  buf.at[pl.ds(CH, CH)], sem)
            d0.wait()
            d1.wait()
            p0 = pltpu.async_copy(buf.at[pl.ds(0, CH)],
                                  acc.at[dst_v.at[i0]], sem2, add=True)
            p1 = pltpu.async_copy(buf.at[pl.ds(CH, CH)],
                                  acc.at[dst_v.at[i0 + 1]], sem2, add=True)
            p0.wait()
            p1.wait()
            return carry2

        lax.fori_loop(0, IB3 // 2, group, 0)
        return carry

    lax.fori_loop(0, C3 // IB3, block, 0)
    plsc.subcore_barrier()
    pltpu.sync_copy(acc.at[pl.ds(base, RPT)], s_out.at[c, pl.ds(base, RPT)])


_gcn_call = pl.kernel(
    _gcn_body,
    out_type=jax.ShapeDtypeStruct((2, NPAD, 128), _f32),
    mesh=_mesh(),
    scratch_types=[
        pltpu.VMEM((IB3, CH), jnp.int32),
        pltpu.VMEM((IB3, CH), jnp.int32),
        pltpu.VMEM((2 * CH, 128), _f32),
        pltpu.VMEM_SHARED((NPAD, 128), _f32),
        pltpu.SemaphoreType.DMA,
        pltpu.SemaphoreType.DMA,
    ],
)


def _dense_body(x_ref, agg_ref, W1_ref, b1_ref, g_ref, be_ref,
                W2_ref, b2_ref, Wc_ref, y_ref, dinv_ref):
    x = x_ref[...]
    a = x + agg_ref[0, :N, :DIN] + agg_ref[1, :N, :DIN]
    h = jnp.dot(a, W1_ref[...], preferred_element_type=_f32) + b1_ref[...]
    m = jnp.mean(h, axis=0, keepdims=True)
    d0 = h - m
    v = jnp.mean(d0 * d0, axis=0, keepdims=True)
    h = d0 * lax.rsqrt(v + 1e-5) * g_ref[...] + be_ref[...]
    h = jnp.maximum(h, 0.0)
    h = jnp.maximum(jnp.dot(h, W2_ref[...], preferred_element_type=_f32)
                    + b2_ref[...], 0.0)
    xw = jnp.dot(h, Wc_ref[...], preferred_element_type=_f32)
    dg = (agg_ref[0, :N, DIN:DIN + 1] + agg_ref[1, :N, DIN:DIN + 1]) + 1.0
    dinv = lax.rsqrt(dg)
    y = xw * dinv
    y_ref[0, :N, :] = y[:, :128]
    y_ref[1, :N, :] = y[:, 128:]
    dinv_ref[...] = jnp.broadcast_to(dinv, (N, 128))


def _dense_call(x, agg, W1, b1, gamma, beta, W2, b2, Wc):
    return pl.pallas_call(
        _dense_body,
        out_shape=[jax.ShapeDtypeStruct((2, NPAD, 128), _f32),
                   jax.ShapeDtypeStruct((N, 128), _f32)],
    )(x, agg, W1, b1, gamma, beta, W2, b2, Wc)


def _final_body(s_ref, y_ref, dinv_ref, bmu_ref, bls_ref, mu_ref, ls_ref):
    dinv = dinv_ref[...]
    mu_ref[...] = (s_ref[0, :N, :] + y_ref[0, :N, :]) * dinv + bmu_ref[...]
    ls_ref[...] = (s_ref[1, :N, :] + y_ref[1, :N, :]) * dinv + bls_ref[...]


def _final_call(s_out, ycat, dinvb, bmu, bls):
    return pl.pallas_call(
        _final_body,
        out_shape=[jax.ShapeDtypeStruct((N, 128), _f32),
                   jax.ShapeDtypeStruct((N, 128), _f32)],
    )(s_out, ycat, dinvb, bmu, bls)


def kernel(x, edge_index, W1, b1, gamma, beta, W2, b2, Wmu, bmu, Wls, bls):
    src = edge_index[0]
    dst = edge_index[1]
    npad = EPAD - E
    ar = jnp.arange(npad, dtype=jnp.int32)
    src_p = jnp.concatenate([src, (ar * 997) % N])
    dst_p = jnp.concatenate([dst, N + (ar % (NPAD - N))])
    srcb1 = src_p.reshape(32, C1, CH)
    dstb1 = dst_p.reshape(32, C1, CH)
    src3 = jnp.stack([src_p, src_p + NPAD]).reshape(2, 16, C3, CH)
    dst3 = dst_p.reshape(16, C3, CH)
    zrows = jnp.zeros((RPT, 128), _f32)
    zrowse = jnp.zeros((RPT, DE), _f32)
    xe = jnp.concatenate(
        [x, jnp.ones((N, 1), _f32), jnp.zeros((N, DE - DIN - 1), _f32)],
        axis=1)

    agg = _gin_call(xe, srcb1, dstb1, zrowse)
    ycat, dinvb = _dense_call(
        x, agg, W1, b1.reshape(1, DH), gamma.reshape(1, DH),
        beta.reshape(1, DH), W2, b2.reshape(1, DH),
        jnp.concatenate([Wmu, Wls], axis=1))
    s_out = _gcn_call(ycat.reshape(2 * NPAD, 128), src3, dst3, zrows)
    mu, ls = _final_call(s_out, ycat, dinvb,
                         bmu.reshape(1, 128), bls.reshape(1, 128))
    return (mu, ls)


# speedup vs baseline: 17.6461x; 1.0213x over previous
import jax
import jax.numpy as jnp
from jax import lax
from jax.experimental import pallas as pl
from jax.experimental.pallas import tpu as pltpu
from jax.experimental.pallas import tpu_sc as plsc

N = 10000
DIN = 128
DH = 256
NPAD = 10112
RPT = NPAD // 16
E = 320000
CH = 128
CH1 = 64
EPAD = 327680
C1 = EPAD // (32 * CH1)
C3 = EPAD // (16 * CH)
IB1 = 32
IB3 = 40

_f32 = jnp.float32


def _mesh():
    return plsc.VectorSubcoreMesh(core_axis_name="c", subcore_axis_name="s")


def _pipelined_block(table, src_v, dst_v, acc, b0, b1,
                     sg0, sg1, ss0, ss1, nch):
    def pair(j, carry):
        i0 = 2 * j
        g0 = pltpu.async_copy(table.at[src_v.at[i0]], b0, sg0)
        g1 = pltpu.async_copy(table.at[src_v.at[i0 + 1]], b1, sg1)
        g0.wait()
        s0 = pltpu.async_copy(b0, acc.at[dst_v.at[i0]], ss0, add=True)
        g1.wait()
        s1 = pltpu.async_copy(b1, acc.at[dst_v.at[i0 + 1]], ss1, add=True)
        s0.wait()
        s1.wait()
        return carry

    lax.fori_loop(0, nch // 2, pair, 0)


DE = 144


def _gin_body(xe_hbm, srcb, dstb, zr, agg_out,
              src_v, dst_v, b0, b1, acc, sg0, sg1, ss0, ss1):
    c = lax.axis_index("c")
    s = lax.axis_index("s")
    wid = s * 2 + c
    base = s * RPT
    pltpu.sync_copy(zr, acc.at[pl.ds(base, RPT)])
    plsc.subcore_barrier()

    def block(bk, carry):
        pltpu.sync_copy(srcb.at[wid, pl.ds(bk * IB1, IB1)], src_v)
        pltpu.sync_copy(dstb.at[wid, pl.ds(bk * IB1, IB1)], dst_v)
        _pipelined_block(xe_hbm, src_v, dst_v, acc, b0, b1,
                         sg0, sg1, ss0, ss1, IB1)
        return carry

    lax.fori_loop(0, C1 // IB1, block, 0)
    plsc.subcore_barrier()
    pltpu.sync_copy(acc.at[pl.ds(base, RPT)], agg_out.at[c, pl.ds(base, RPT)])


_gin_call = pl.kernel(
    _gin_body,
    out_type=jax.ShapeDtypeStruct((2, NPAD, DE), _f32),
    mesh=_mesh(),
    scratch_types=[
        pltpu.VMEM((IB1, CH1), jnp.int32),
        pltpu.VMEM((IB1, CH1), jnp.int32),
        pltpu.VMEM((CH1, DE), _f32),
        pltpu.VMEM((CH1, DE), _f32),
        pltpu.VMEM_SHARED((NPAD, DE), _f32),
        pltpu.SemaphoreType.DMA,
        pltpu.SemaphoreType.DMA,
        pltpu.SemaphoreType.DMA,
        pltpu.SemaphoreType.DMA,
    ],
    compiler_params=pltpu.CompilerParams(use_tc_tiling_on_sc=False),
)


def _gcn_body(y_hbm, srcb, dstb, zr, s_out,
              src_v, dst_v, b0, b1, acc, sg0, sg1, ss0, ss1):
    c = lax.axis_index("c")
    s = lax.axis_index("s")
    base = s * RPT
    pltpu.sync_copy(zr, acc.at[pl.ds(base, RPT)])
    plsc.subcore_barrier()

    def block(bk, carry):
        pltpu.sync_copy(srcb.at[c, s, pl.ds(bk * IB3, IB3)], src_v)
        pltpu.sync_copy(dstb.at[s, pl.ds(bk * IB3, IB3)], dst_v)
        _pipelined_block(y_hbm, src_v, dst_v, acc, b0, b1,
                         sg0, sg1, ss0, ss1, IB3)
        return carry

    lax.fori_loop(0, C3 // IB3, block, 0)
    plsc.subcore_barrier()
    pltpu.sync_copy(acc.at[pl.ds(base, RPT)], s_out.at[c, pl.ds(base, RPT)])


_gcn_call = pl.kernel(
    _gcn_body,
    out_type=jax.ShapeDtypeStruct((2, NPAD, 128), _f32),
    mesh=_mesh(),
    scratch_types=[
        pltpu.VMEM((IB3, CH), jnp.int32),
        pltpu.VMEM((IB3, CH), jnp.int32),
        pltpu.VMEM((CH, 128), _f32),
        pltpu.VMEM((CH, 128), _f32),
        pltpu.VMEM_SHARED((NPAD, 128), _f32),
        pltpu.SemaphoreType.DMA,
        pltpu.SemaphoreType.DMA,
        pltpu.SemaphoreType.DMA,
        pltpu.SemaphoreType.DMA,
    ],
)


def _dense_body(x_ref, agg_ref, W1_ref, b1_ref, g_ref, be_ref,
                W2_ref, b2_ref, Wc_ref, y_ref, dinv_ref):
    x = x_ref[...]
    a = x + agg_ref[0, :N, :DIN] + agg_ref[1, :N, :DIN]
    h = jnp.dot(a, W1_ref[...], preferred_element_type=_f32) + b1_ref[...]
    m = jnp.mean(h, axis=0, keepdims=True)
    d0 = h - m
    v = jnp.mean(d0 * d0, axis=0, keepdims=True)
    h = d0 * lax.rsqrt(v + 1e-5) * g_ref[...] + be_ref[...]
    h = jnp.maximum(h, 0.0)
    h = jnp.maximum(jnp.dot(h, W2_ref[...], preferred_element_type=_f32)
                    + b2_ref[...], 0.0)
    xw = jnp.dot(h, Wc_ref[...], preferred_element_type=_f32)
    dg = (agg_ref[0, :N, DIN:DIN + 1] + agg_ref[1, :N, DIN:DIN + 1]) + 1.0
    dinv = lax.rsqrt(dg)
    y = xw * dinv
    y_ref[0, :N, :] = y[:, :128]
    y_ref[1, :N, :] = y[:, 128:]
    dinv_ref[...] = jnp.broadcast_to(dinv, (N, 128))


def _dense_call(x, agg, W1, b1, gamma, beta, W2, b2, Wc):
    return pl.pallas_call(
        _dense_body,
        out_shape=[jax.ShapeDtypeStruct((2, NPAD, 128), _f32),
                   jax.ShapeDtypeStruct((N, 128), _f32)],
    )(x, agg, W1, b1, gamma, beta, W2, b2, Wc)


def _final_body(s_ref, y_ref, dinv_ref, bmu_ref, bls_ref, mu_ref, ls_ref):
    dinv = dinv_ref[...]
    mu_ref[...] = (s_ref[0, :N, :] + y_ref[0, :N, :]) * dinv + bmu_ref[...]
    ls_ref[...] = (s_ref[1, :N, :] + y_ref[1, :N, :]) * dinv + bls_ref[...]


def _final_call(s_out, ycat, dinvb, bmu, bls):
    return pl.pallas_call(
        _final_body,
        out_shape=[jax.ShapeDtypeStruct((N, 128), _f32),
                   jax.ShapeDtypeStruct((N, 128), _f32)],
    )(s_out, ycat, dinvb, bmu, bls)


def kernel(x, edge_index, W1, b1, gamma, beta, W2, b2, Wmu, bmu, Wls, bls):
    src = edge_index[0]
    dst = edge_index[1]
    npad = EPAD - E
    ar = jnp.arange(npad, dtype=jnp.int32)
    src_p = jnp.concatenate([src, (ar * 997) % N])
    dst_p = jnp.concatenate([dst, N + (ar % (NPAD - N))])
    srcb1 = src_p.reshape(32, C1, CH1)
    dstb1 = dst_p.reshape(32, C1, CH1)
    src3 = jnp.stack([src_p, src_p + NPAD]).reshape(2, 16, C3, CH)
    dst3 = dst_p.reshape(16, C3, CH)
    zrows = jnp.zeros((RPT, 128), _f32)
    zrowse = jnp.zeros((RPT, DE), _f32)
    xe = jnp.concatenate(
        [x, jnp.ones((N, 1), _f32), jnp.zeros((N, DE - DIN - 1), _f32)],
        axis=1)

    agg = _gin_call(xe, srcb1, dstb1, zrowse)
    ycat, dinvb = _dense_call(
        x, agg, W1, b1.reshape(1, DH), gamma.reshape(1, DH),
        beta.reshape(1, DH), W2, b2.reshape(1, DH),
        jnp.concatenate([Wmu, Wls], axis=1))
    s_out = _gcn_call(ycat.reshape(2 * NPAD, 128), src3, dst3, zrows)
    mu, ls = _final_call(s_out, ycat, dinvb,
                         bmu.reshape(1, 128), bls.reshape(1, 128))
    return (mu, ls)


# speedup vs baseline: 20.0974x; 1.1389x over previous
import jax
import jax.numpy as jnp
from jax import lax
from jax.experimental import pallas as pl
from jax.experimental.pallas import tpu as pltpu
from jax.experimental.pallas import tpu_sc as plsc

N = 10000
DIN = 128
DH = 256
NPAD = 10112
RPT = NPAD // 16
E = 320000
CH = 128
CH1 = 64
EPAD = 327680
C1 = EPAD // (32 * CH1)
C3 = EPAD // (16 * CH)
IB1 = 32
IB3 = 40

_f32 = jnp.float32


def _mesh():
    return plsc.VectorSubcoreMesh(core_axis_name="c", subcore_axis_name="s")


def _pipelined_block(table, src_v, dst_v, acc, b0, b1,
                     sg0, sg1, ss0, ss1, nch):
    def pair(j, carry):
        i0 = 2 * j
        g0 = pltpu.async_copy(table.at[src_v.at[i0]], b0, sg0)
        g1 = pltpu.async_copy(table.at[src_v.at[i0 + 1]], b1, sg1)
        g0.wait()
        s0 = pltpu.async_copy(b0, acc.at[dst_v.at[i0]], ss0, add=True)
        g1.wait()
        s1 = pltpu.async_copy(b1, acc.at[dst_v.at[i0 + 1]], ss1, add=True)
        s0.wait()
        s1.wait()
        return carry

    lax.fori_loop(0, nch // 2, pair, 0)


DE = 144


def _gin_body(xe_hbm, srcb, dstb, zr, agg_out,
              src_v, dst_v, b0, b1, acc, sg0, sg1, ss0, ss1):
    c = lax.axis_index("c")
    s = lax.axis_index("s")
    wid = s * 2 + c
    base = s * RPT
    pltpu.sync_copy(zr, acc.at[pl.ds(base, RPT)])
    plsc.subcore_barrier()

    def block(bk, carry):
        pltpu.sync_copy(srcb.at[wid, pl.ds(bk * IB1, IB1)], src_v)
        pltpu.sync_copy(dstb.at[wid, pl.ds(bk * IB1, IB1)], dst_v)
        _pipelined_block(xe_hbm, src_v, dst_v, acc, b0, b1,
                         sg0, sg1, ss0, ss1, IB1)
        return carry

    lax.fori_loop(0, C1 // IB1, block, 0)
    plsc.subcore_barrier()
    pltpu.sync_copy(acc.at[pl.ds(base, RPT)], agg_out.at[c, pl.ds(base, RPT)])


_gin_call = pl.kernel(
    _gin_body,
    out_type=jax.ShapeDtypeStruct((2, NPAD, DE), _f32),
    mesh=_mesh(),
    scratch_types=[
        pltpu.VMEM((IB1, CH1), jnp.int32),
        pltpu.VMEM((IB1, CH1), jnp.int32),
        pltpu.VMEM((CH1, DE), _f32),
        pltpu.VMEM((CH1, DE), _f32),
        pltpu.VMEM_SHARED((NPAD, DE), _f32),
        pltpu.SemaphoreType.DMA,
        pltpu.SemaphoreType.DMA,
        pltpu.SemaphoreType.DMA,
        pltpu.SemaphoreType.DMA,
    ],
    compiler_params=pltpu.CompilerParams(use_tc_tiling_on_sc=False),
)


def _gcn_body(y_hbm, srcb, dstb, zr, s_out,
              src_v, dst_v, b0, b1, acc, sg0, sg1, ss0, ss1):
    c = lax.axis_index("c")
    s = lax.axis_index("s")
    base = s * RPT
    pltpu.sync_copy(zr, acc.at[pl.ds(base, RPT)])
    plsc.subcore_barrier()

    def block(bk, carry):
        pltpu.sync_copy(srcb.at[c, s, pl.ds(bk * IB3, IB3)], src_v)
        pltpu.sync_copy(dstb.at[s, pl.ds(bk * IB3, IB3)], dst_v)
        _pipelined_block(y_hbm, src_v, dst_v, acc, b0, b1,
                         sg0, sg1, ss0, ss1, IB3)
        return carry

    lax.fori_loop(0, C3 // IB3, block, 0)
    plsc.subcore_barrier()
    pltpu.sync_copy(acc.at[pl.ds(base, RPT)], s_out.at[c, pl.ds(base, RPT)])


_bf16 = jnp.bfloat16

_gcn_call = pl.kernel(
    _gcn_body,
    out_type=jax.ShapeDtypeStruct((2, NPAD, 128), _bf16),
    mesh=_mesh(),
    scratch_types=[
        pltpu.VMEM((IB3, CH), jnp.int32),
        pltpu.VMEM((IB3, CH), jnp.int32),
        pltpu.VMEM((CH, 128), _bf16),
        pltpu.VMEM((CH, 128), _bf16),
        pltpu.VMEM_SHARED((NPAD, 128), _bf16),
        pltpu.SemaphoreType.DMA,
        pltpu.SemaphoreType.DMA,
        pltpu.SemaphoreType.DMA,
        pltpu.SemaphoreType.DMA,
    ],
    compiler_params=pltpu.CompilerParams(use_tc_tiling_on_sc=False),
)


def _dense_body(x_ref, agg_ref, W1_ref, b1_ref, g_ref, be_ref,
                W2_ref, b2_ref, Wc_ref, y_ref, dinv_ref):
    x = x_ref[...]
    a = x + agg_ref[0, :N, :DIN] + agg_ref[1, :N, :DIN]
    h = jnp.dot(a, W1_ref[...], preferred_element_type=_f32) + b1_ref[...]
    m = jnp.mean(h, axis=0, keepdims=True)
    d0 = h - m
    v = jnp.mean(d0 * d0, axis=0, keepdims=True)
    h = d0 * lax.rsqrt(v + 1e-5) * g_ref[...] + be_ref[...]
    h = jnp.maximum(h, 0.0)
    h = jnp.maximum(jnp.dot(h, W2_ref[...], preferred_element_type=_f32)
                    + b2_ref[...], 0.0)
    xw = jnp.dot(h, Wc_ref[...], preferred_element_type=_f32)
    dg = (agg_ref[0, :N, DIN:DIN + 1] + agg_ref[1, :N, DIN:DIN + 1]) + 1.0
    dinv = lax.rsqrt(dg)
    y = (xw * dinv).astype(jnp.bfloat16)
    y_ref[0, :N, :] = y[:, :128]
    y_ref[1, :N, :] = y[:, 128:]
    dinv_ref[...] = jnp.broadcast_to(dinv, (N, 128))


def _dense_call(x, agg, W1, b1, gamma, beta, W2, b2, Wc):
    return pl.pallas_call(
        _dense_body,
        out_shape=[jax.ShapeDtypeStruct((2, NPAD, 128), jnp.bfloat16),
                   jax.ShapeDtypeStruct((N, 128), _f32)],
    )(x, agg, W1, b1, gamma, beta, W2, b2, Wc)


def _final_body(s_ref, y_ref, dinv_ref, bmu_ref, bls_ref, mu_ref, ls_ref):
    dinv = dinv_ref[...]
    mu_ref[...] = ((s_ref[0, :N, :].astype(_f32)
                    + y_ref[0, :N, :].astype(_f32)) * dinv + bmu_ref[...])
    ls_ref[...] = ((s_ref[1, :N, :].astype(_f32)
                    + y_ref[1, :N, :].astype(_f32)) * dinv + bls_ref[...])


def _final_call(s_out, ycat, dinvb, bmu, bls):
    return pl.pallas_call(
        _final_body,
        out_shape=[jax.ShapeDtypeStruct((N, 128), _f32),
                   jax.ShapeDtypeStruct((N, 128), _f32)],
    )(s_out, ycat, dinvb, bmu, bls)


def kernel(x, edge_index, W1, b1, gamma, beta, W2, b2, Wmu, bmu, Wls, bls):
    src = edge_index[0]
    dst = edge_index[1]
    npad = EPAD - E
    ar = jnp.arange(npad, dtype=jnp.int32)
    src_p = jnp.concatenate([src, (ar * 997) % N])
    dst_p = jnp.concatenate([dst, N + (ar % (NPAD - N))])
    srcb1 = src_p.reshape(32, C1, CH1)
    dstb1 = dst_p.reshape(32, C1, CH1)
    src3 = jnp.stack([src_p, src_p + NPAD]).reshape(2, 16, C3, CH)
    dst3 = dst_p.reshape(16, C3, CH)
    zrows = jnp.zeros((RPT, 128), jnp.bfloat16)
    zrowse = jnp.zeros((RPT, DE), _f32)
    xe = jnp.concatenate(
        [x, jnp.ones((N, 1), _f32), jnp.zeros((N, DE - DIN - 1), _f32)],
        axis=1)

    agg = _gin_call(xe, srcb1, dstb1, zrowse)
    ycat, dinvb = _dense_call(
        x, agg, W1, b1.reshape(1, DH), gamma.reshape(1, DH),
        beta.reshape(1, DH), W2, b2.reshape(1, DH),
        jnp.concatenate([Wmu, Wls], axis=1))
    s_out = _gcn_call(ycat.reshape(2 * NPAD, 128), src3, dst3, zrows)
    mu, ls = _final_call(s_out, ycat, dinvb,
                         bmu.reshape(1, 128), bls.reshape(1, 128))
    return (mu, ls)
